# baseline (speedup 1.0000x reference)
import functools

import jax
import jax.numpy as jnp
from jax import lax
from jax.experimental import pallas as pl
from jax.experimental.pallas import tpu as pltpu
from jax.experimental.pallas import tpu_sc as plsc

_N = 10000
_E = 320000
_D = 128

_NC = 2
_NS = 16
_NW = _NC * _NS
_EPW = _E // _NW
_C = 100
_NCHUNK = _EPW // _C
_NBUF = 3
_NFULL = (_NCHUNK // _NBUF) * _NBUF
_NP = 10112
_RPT = _NP // _NS
_XT = _N - (_NS - 1) * _RPT


def _sc_aggregate(sd, x, zrows):
    mesh = plsc.VectorSubcoreMesh(core_axis_name="c", subcore_axis_name="s")

    @functools.partial(
        pl.kernel,
        mesh=mesh,
        out_type=jax.ShapeDtypeStruct((_NC, _NP, _D), jnp.float32),
        scratch_types=[
            [pltpu.VMEM((2, _C), jnp.int32)] * _NBUF,
            [pltpu.VMEM((_C, _D), jnp.float32)] * _NBUF,
            pltpu.VMEM_SHARED((_NP, _D), jnp.float32),
            [pltpu.SemaphoreType.DMA] * _NBUF,
            [pltpu.SemaphoreType.DMA] * _NBUF,
            [pltpu.SemaphoreType.DMA] * _NBUF,
        ],
    )
    def agg_kernel(sd_hbm, x_hbm, z_hbm, out_hbm,
                   idxb, rows, agg_s, sem_i, sem_g, sem_s):
        c = lax.axis_index("c")
        s = lax.axis_index("s")
        w = c * _NS + s

        def idx_start(k, b):
            pltpu.async_copy(sd_hbm.at[:, w * _NCHUNK + k], idxb[b], sem_i[b])

        def idx_wait(k, b):
            pltpu.make_async_copy(
                sd_hbm.at[:, w * _NCHUNK + k], idxb[b], sem_i[b]).wait()

        def gather_start(b):
            pltpu.async_copy(x_hbm.at[idxb[b].at[0]], rows[b], sem_g[b])

        def gather_wait(b):
            pltpu.make_async_copy(
                x_hbm.at[idxb[b].at[0]], rows[b], sem_g[b]).wait()

        def scatter_start(b):
            pltpu.async_copy(
                rows[b], agg_s.at[idxb[b].at[1]], sem_s[b], add=True)

        def scatter_wait(b):
            pltpu.make_async_copy(
                rows[b], agg_s.at[idxb[b].at[1]], sem_s[b]).wait()

        for b in range(_NBUF):
            idx_start(b, b)
        base = s * _RPT

        @pl.when(jnp.logical_and(c == 0, s < _NS - 1))
        def _():
            pltpu.sync_copy(x_hbm.at[pl.ds(base, _RPT)],
                            agg_s.at[pl.ds(base, _RPT)])

        @pl.when(jnp.logical_and(c == 0, s == _NS - 1))
        def _():
            pltpu.sync_copy(x_hbm.at[pl.ds(_N - _XT, _XT)],
                            agg_s.at[pl.ds(_N - _XT, _XT)])
            pltpu.sync_copy(z_hbm.at[pl.ds(0, _NP - _N)],
                            agg_s.at[pl.ds(_N, _NP - _N)])

        @pl.when(c == 1)
        def _():
            pltpu.sync_copy(z_hbm, agg_s.at[pl.ds(base, _RPT)])
        for b in range(_NBUF):
            idx_wait(b, b)
            gather_start(b)
        plsc.subcore_barrier()

        @pl.loop(0, _NFULL, step=_NBUF)
        def _(j):
            for b in range(_NBUF):
                k = j + b
                gather_wait(b)
                scatter_start(b)

                @pl.when(k + _NBUF < _NCHUNK)
                def _():
                    idx_start(k + _NBUF, b)
                    scatter_wait(b)
                    idx_wait(k + _NBUF, b)
                    gather_start(b)

        for b in range(_NCHUNK - _NFULL):
            gather_wait(b)
            scatter_start(b)
        for b in range(_NBUF):
            scatter_wait(b)

        plsc.subcore_barrier()
        pltpu.sync_copy(agg_s.at[pl.ds(s * _RPT, _RPT)],
                        out_hbm.at[c, pl.ds(s * _RPT, _RPT)])

    return agg_kernel(sd, x, zrows)


def _mlp_body(agg_ref, w1_ref, g1_ref, be1_ref,
              w2_ref, g2_ref, be2_ref, out_ref):
    def bn_relu(h, g_ref, be_ref):
        mu = jnp.mean(h, axis=0, keepdims=True)
        m2 = jnp.mean(jnp.square(h), axis=0, keepdims=True)
        var = m2 - jnp.square(mu)
        scale = lax.rsqrt(var + 1e-5) * g_ref[...]
        return jnp.maximum(h * scale + (be_ref[...] - mu * scale), 0.0)

    h = agg_ref[0, :_N] + agg_ref[1, :_N]
    h = bn_relu(jnp.dot(h, w1_ref[...], preferred_element_type=jnp.float32),
                g1_ref, be1_ref)
    out_ref[...] = bn_relu(
        jnp.dot(h, w2_ref[...], preferred_element_type=jnp.float32),
        g2_ref, be2_ref)


def _tc_mlp(agg, W1, g1, be1, W2, g2, be2):
    return pl.pallas_call(
        _mlp_body,
        out_shape=jax.ShapeDtypeStruct((_N, _D), jnp.float32),
    )(agg, W1, g1, be1, W2, g2, be2)


def kernel(x, pos, edge_index, W1, b1, g1, be1, W2, b2, g2, be2):
    sd = edge_index.reshape(2, _NW * _NCHUNK, _C)
    zrows = jnp.zeros((_RPT, _D), jnp.float32)
    agg = _sc_aggregate(sd, x, zrows)
    row = lambda v: v.reshape(1, _D)
    return _tc_mlp(agg, W1, row(g1), row(be1), W2, row(g2), row(be2))


# speedup vs baseline: 1.0073x; 1.0073x over previous
import functools

import jax
import jax.numpy as jnp
from jax import lax
from jax.experimental import pallas as pl
from jax.experimental.pallas import tpu as pltpu
from jax.experimental.pallas import tpu_sc as plsc

_N = 10000
_E = 320000
_D = 128

_NC = 2
_NS = 16
_NW = _NC * _NS
_EPW = _E // _NW
_C = 100
_NCHUNK = _EPW // _C
_NBUF = 3
_NFULL = (_NCHUNK // _NBUF) * _NBUF
_NP = 10112
_RPT = _NP // _NS
_XT = _N - (_NS - 1) * _RPT


def _sc_aggregate(sd, x, zrows):
    mesh = plsc.VectorSubcoreMesh(core_axis_name="c", subcore_axis_name="s")

    @functools.partial(
        pl.kernel,
        mesh=mesh,
        out_type=jax.ShapeDtypeStruct((_NC, _NP, _D), jnp.float32),
        scratch_types=[
            [pltpu.VMEM((2, _C), jnp.int32)] * _NBUF,
            [pltpu.VMEM((_C, _D), jnp.float32)] * _NBUF,
            pltpu.VMEM_SHARED((_NP, _D), jnp.float32),
            [pltpu.SemaphoreType.DMA] * _NBUF,
            [pltpu.SemaphoreType.DMA] * _NBUF,
            [pltpu.SemaphoreType.DMA] * _NBUF,
        ],
    )
    def agg_kernel(sd_hbm, x_hbm, z_hbm, out_hbm,
                   idxb, rows, agg_s, sem_i, sem_g, sem_s):
        c = lax.axis_index("c")
        s = lax.axis_index("s")
        w = c * _NS + s

        def idx_start(k, b):
            pltpu.async_copy(sd_hbm.at[w, k], idxb[b], sem_i[b])

        def idx_wait(k, b):
            pltpu.make_async_copy(sd_hbm.at[w, k], idxb[b], sem_i[b]).wait()

        def gather_start(b):
            pltpu.async_copy(x_hbm.at[idxb[b].at[0]], rows[b], sem_g[b])

        def gather_wait(b):
            pltpu.make_async_copy(
                x_hbm.at[idxb[b].at[0]], rows[b], sem_g[b]).wait()

        def scatter_start(b):
            pltpu.async_copy(
                rows[b], agg_s.at[idxb[b].at[1]], sem_s[b], add=True)

        def scatter_wait(b):
            pltpu.make_async_copy(
                rows[b], agg_s.at[idxb[b].at[1]], sem_s[b]).wait()

        for b in range(_NBUF):
            idx_start(b, b)
        base = s * _RPT

        @pl.when(jnp.logical_and(c == 0, s < _NS - 1))
        def _():
            pltpu.sync_copy(x_hbm.at[pl.ds(base, _RPT)],
                            agg_s.at[pl.ds(base, _RPT)])

        @pl.when(jnp.logical_and(c == 0, s == _NS - 1))
        def _():
            pltpu.sync_copy(x_hbm.at[pl.ds(_N - _XT, _XT)],
                            agg_s.at[pl.ds(_N - _XT, _XT)])
            pltpu.sync_copy(z_hbm.at[pl.ds(0, _NP - _N)],
                            agg_s.at[pl.ds(_N, _NP - _N)])

        @pl.when(c == 1)
        def _():
            pltpu.sync_copy(z_hbm, agg_s.at[pl.ds(base, _RPT)])
        for b in range(_NBUF):
            idx_wait(b, b)
            gather_start(b)
        plsc.subcore_barrier()

        @pl.loop(0, _NFULL, step=_NBUF)
        def _(j):
            for b in range(_NBUF):
                k = j + b
                gather_wait(b)
                scatter_start(b)

                @pl.when(k + _NBUF < _NCHUNK)
                def _():
                    idx_start(k + _NBUF, b)
                    scatter_wait(b)
                    idx_wait(k + _NBUF, b)
                    gather_start(b)

        for b in range(_NCHUNK - _NFULL):
            gather_wait(b)
            scatter_start(b)
        for b in range(_NBUF):
            scatter_wait(b)

        plsc.subcore_barrier()
        pltpu.sync_copy(agg_s.at[pl.ds(s * _RPT, _RPT)],
                        out_hbm.at[c, pl.ds(s * _RPT, _RPT)])

    return agg_kernel(sd, x, zrows)


def _mlp_body(agg_ref, w1_ref, g1_ref, be1_ref,
              w2_ref, g2_ref, be2_ref, out_ref):
    def bn_relu(h, g_ref, be_ref):
        mu = jnp.mean(h, axis=0, keepdims=True)
        m2 = jnp.mean(jnp.square(h), axis=0, keepdims=True)
        var = m2 - jnp.square(mu)
        scale = lax.rsqrt(var + 1e-5) * g_ref[...]
        return jnp.maximum(h * scale + (be_ref[...] - mu * scale), 0.0)

    h = agg_ref[0, :_N] + agg_ref[1, :_N]
    h = bn_relu(jnp.dot(h, w1_ref[...], preferred_element_type=jnp.float32),
                g1_ref, be1_ref)
    out_ref[...] = bn_relu(
        jnp.dot(h, w2_ref[...], preferred_element_type=jnp.float32),
        g2_ref, be2_ref)


def _tc_mlp(agg, W1, g1, be1, W2, g2, be2):
    return pl.pallas_call(
        _mlp_body,
        out_shape=jax.ShapeDtypeStruct((_N, _D), jnp.float32),
    )(agg, W1, g1, be1, W2, g2, be2)


def kernel(x, pos, edge_index, W1, b1, g1, be1, W2, b2, g2, be2):
    sd = edge_index.reshape(2, _NW, _NCHUNK, _C).transpose(1, 2, 0, 3)
    zrows = jnp.zeros((_RPT, _D), jnp.float32)
    agg = _sc_aggregate(sd, x, zrows)
    row = lambda v: v.reshape(1, _D)
    return _tc_mlp(agg, W1, row(g1), row(be1), W2, row(g2), row(be2))
